# Initial kernel scaffold; baseline (speedup 1.0000x reference)
import jax
import jax.numpy as jnp
from jax.experimental import pallas as pl


def kernel(x, edge_index, W1, a1s, a1d, b1, g1, be1, W2, a2s, a2d, b2, g2, be2, W3, a3s, a3d, b3):
    raise NotImplementedError("write your pallas kernel here")


# speedup vs baseline: 11.5254x; 11.5254x over previous
import functools
import jax
import jax.numpy as jnp
from jax import lax
from jax.experimental import pallas as pl
from jax.experimental.pallas import tpu as pltpu
from jax.experimental.pallas import tpu_sc as plsc

NW = 32
BLK = 128
ROWB = 1000


def _pre_body(x_ref, w_ref, as_ref, ad_ref, h_ref, ps_ref, pd_ref):
    hb = jnp.dot(x_ref[...], w_ref[...], preferred_element_type=jnp.float32)
    h_ref[...] = hb
    ps_ref[...] = jnp.dot(hb, as_ref[...], preferred_element_type=jnp.float32)
    pd_ref[...] = jnp.dot(hb, ad_ref[...], preferred_element_type=jnp.float32)


def _pre(x, W, as128, ad128):
    N, Din = x.shape
    F = W.shape[1]
    grid = (N // ROWB,)
    return pl.pallas_call(
        _pre_body,
        grid=grid,
        in_specs=[
            pl.BlockSpec((ROWB, Din), lambda i: (i, 0)),
            pl.BlockSpec((Din, F), lambda i: (0, 0)),
            pl.BlockSpec((F, 128), lambda i: (0, 0)),
            pl.BlockSpec((F, 128), lambda i: (0, 0)),
        ],
        out_specs=[
            pl.BlockSpec((ROWB, F), lambda i: (i, 0)),
            pl.BlockSpec((ROWB, 128), lambda i: (i, 0)),
            pl.BlockSpec((ROWB, 128), lambda i: (i, 0)),
        ],
        out_shape=[
            jax.ShapeDtypeStruct((N, F), jnp.float32),
            jax.ShapeDtypeStruct((N, 128), jnp.float32),
            jax.ShapeDtypeStruct((N, 128), jnp.float32),
        ],
    )(x, W, as128, ad128)


def _post_body_ln(n0_ref, n1_ref, d0_ref, d1_ref, r_ref, b_ref, g_ref,
                  be_ref, o_ref):
    num = n0_ref[...] + n1_ref[...]
    den = jnp.dot(d0_ref[...] + d1_ref[...], r_ref[...],
                  preferred_element_type=jnp.float32)
    y = num / (den + 1e-16) + b_ref[...]
    m = jnp.mean(y, axis=-1, keepdims=True)
    d = y - m
    v = jnp.mean(d * d, axis=-1, keepdims=True)
    o_ref[...] = jnp.maximum(d / jnp.sqrt(v + 1e-5) * g_ref[...] + be_ref[...],
                             0.0)


def _post_body_plain(n0_ref, n1_ref, d0_ref, d1_ref, r_ref, b_ref, o_ref):
    num = n0_ref[...] + n1_ref[...]
    den = jnp.dot(d0_ref[...] + d1_ref[...], r_ref[...],
                  preferred_element_type=jnp.float32)
    o_ref[...] = num / (den + 1e-16) + b_ref[...]


def _post(np0, np1, dp0, dp1, rmat, bias, g=None, be=None):
    N, F = np0.shape
    grid = (N // ROWB,)
    vec = lambda: pl.BlockSpec((1, F), lambda i: (0, 0))
    specs = [
        pl.BlockSpec((ROWB, F), lambda i: (i, 0)),
        pl.BlockSpec((ROWB, F), lambda i: (i, 0)),
        pl.BlockSpec((ROWB, 8), lambda i: (i, 0)),
        pl.BlockSpec((ROWB, 8), lambda i: (i, 0)),
        pl.BlockSpec((8, F), lambda i: (0, 0)),
        vec(),
    ]
    args = [np0, np1, dp0, dp1, rmat, bias.reshape(1, F)]
    if g is not None:
        body = _post_body_ln
        specs += [vec(), vec()]
        args += [g.reshape(1, F), be.reshape(1, F)]
    else:
        body = _post_body_plain
    return pl.pallas_call(
        body,
        grid=grid,
        in_specs=specs,
        out_specs=pl.BlockSpec((ROWB, F), lambda i: (i, 0)),
        out_shape=jax.ShapeDtypeStruct((N, F), jnp.float32),
    )(*args)


@functools.lru_cache(maxsize=None)
def _make_epass(H, N, E_real, EPAD):
    PER_W = EPAD // NW
    NBLK = PER_W // BLK
    R0 = (-(-N // 16) // 8) * 8
    R15 = N - 15 * R0
    mesh = plsc.VectorSubcoreMesh(core_axis_name="c", subcore_axis_name="s")

    @functools.partial(
        pl.kernel,
        out_type=[
            jax.ShapeDtypeStruct((EPAD, 16), jnp.float32),
            jax.ShapeDtypeStruct((2, N, BLK), jnp.float32),
        ],
        mesh=mesh,
        compiler_params=pltpu.CompilerParams(needs_layout_passes=False),
        scratch_types=[
            pltpu.VMEM((BLK,), jnp.int32),
            pltpu.VMEM((BLK,), jnp.int32),
            pltpu.VMEM((BLK, BLK), jnp.float32),
            pltpu.VMEM((BLK, 16), jnp.float32),
            pltpu.VMEM((BLK, BLK), jnp.float32),
            pltpu.VMEM_SHARED((N, BLK), jnp.float32),
            pltpu.SemaphoreType.DMA,
        ],
    )
    def epass(ps_hbm, pd_hbm, src_hbm, dst_hbm, zero_hbm, e_hbm, dpart_hbm,
              sbuf, dbuf, gbuf, e16, erows, dacc, sem1):
        cid = lax.axis_index("c")
        sid = lax.axis_index("s")
        wid = sid * 2 + cid

        def zfill(i, carry):
            for v in range(BLK // 16):
                erows[i, pl.ds(v * 16, 16)] = jnp.zeros((16,), jnp.float32)
            return carry

        lax.fori_loop(0, BLK, zfill, 0)

        @pl.when(sid < 15)
        def _():
            pltpu.sync_copy(zero_hbm.at[pl.ds(sid * R0, R0)],
                            dacc.at[pl.ds(sid * R0, R0)])

        @pl.when(sid == 15)
        def _():
            pltpu.sync_copy(zero_hbm.at[pl.ds(15 * R0, R15)],
                            dacc.at[pl.ds(15 * R0, R15)])

        plsc.subcore_barrier()
        base0 = wid * PER_W
        lane = lax.iota(jnp.int32, 16)
        lmask = lane < H

        def blk(b, carry):
            base = base0 + b * BLK
            pltpu.sync_copy(src_hbm.at[pl.ds(base, BLK)], sbuf)
            pltpu.sync_copy(dst_hbm.at[pl.ds(base, BLK)], dbuf)
            pltpu.async_copy(ps_hbm.at[sbuf], gbuf, sem1).wait()

            def stash(j, c2):
                e16[j, pl.ds(0, 16)] = gbuf[j, pl.ds(0, 16)]
                return c2

            lax.fori_loop(0, BLK, stash, 0)
            pltpu.async_copy(pd_hbm.at[dbuf], gbuf, sem1).wait()

            def edge(j, c2):
                al = e16[j, pl.ds(0, 16)] + gbuf[j, pl.ds(0, 16)]
                al = jnp.where(al > 0, al, 0.2 * al)
                ev = jnp.exp(al)
                keep = jnp.logical_and(lmask, base + j < E_real)
                ev = jnp.where(keep, ev, 0.0)
                e16[j, pl.ds(0, 16)] = ev
                erows[j, pl.ds(0, 16)] = ev
                return c2

            lax.fori_loop(0, BLK, edge, 0)
            pltpu.sync_copy(e16, e_hbm.at[pl.ds(base, BLK), :])
            pltpu.sync_copy(erows, dacc.at[dbuf], add=True)
            return carry

        lax.fori_loop(0, NBLK, blk, 0)
        plsc.subcore_barrier()

        @pl.when(sid < 15)
        def _():
            pltpu.sync_copy(dacc.at[pl.ds(sid * R0, R0)],
                            dpart_hbm.at[cid, pl.ds(sid * R0, R0)])

        @pl.when(sid == 15)
        def _():
            pltpu.sync_copy(dacc.at[pl.ds(15 * R0, R15)],
                            dpart_hbm.at[cid, pl.ds(15 * R0, R15)])

    return epass


@functools.lru_cache(maxsize=None)
def _make_chunk(N, EPAD, head):
    PER_W = EPAD // NW
    NBLK = PER_W // BLK
    R0 = (-(-N // 16) // 8) * 8
    R15 = N - 15 * R0
    mesh = plsc.VectorSubcoreMesh(core_axis_name="c", subcore_axis_name="s")

    @functools.partial(
        pl.kernel,
        out_type=jax.ShapeDtypeStruct((2, N, BLK), jnp.float32),
        mesh=mesh,
        compiler_params=pltpu.CompilerParams(needs_layout_passes=False),
        scratch_types=[
            pltpu.VMEM((BLK,), jnp.int32),
            pltpu.VMEM((BLK,), jnp.int32),
            pltpu.VMEM((BLK, 16), jnp.float32),
            pltpu.VMEM((BLK, BLK), jnp.float32),
            pltpu.VMEM_SHARED((N, BLK), jnp.float32),
            pltpu.SemaphoreType.DMA,
        ],
    )
    def chunk(hc_hbm, e_hbm, src_hbm, dst_hbm, zero_hbm, out_hbm,
              sbuf, dbuf, e16, rows, acc, sem):
        cid = lax.axis_index("c")
        sid = lax.axis_index("s")
        wid = sid * 2 + cid

        @pl.when(sid < 15)
        def _():
            pltpu.sync_copy(zero_hbm.at[pl.ds(sid * R0, R0)],
                            acc.at[pl.ds(sid * R0, R0)])

        @pl.when(sid == 15)
        def _():
            pltpu.sync_copy(zero_hbm.at[pl.ds(15 * R0, R15)],
                            acc.at[pl.ds(15 * R0, R15)])

        plsc.subcore_barrier()
        base0 = wid * PER_W

        def blk(b, carry):
            base = base0 + b * BLK
            pltpu.sync_copy(src_hbm.at[pl.ds(base, BLK)], sbuf)
            pltpu.sync_copy(dst_hbm.at[pl.ds(base, BLK)], dbuf)
            pltpu.sync_copy(e_hbm.at[pl.ds(base, BLK), :], e16)
            pltpu.async_copy(hc_hbm.at[sbuf], rows, sem).wait()

            def srow(j, c2):
                ev = e16[j, pl.ds(0, 16)]
                es = ev[head]
                for v in range(BLK // 16):
                    rows[j, pl.ds(v * 16, 16)] = (
                        rows[j, pl.ds(v * 16, 16)] * es)
                return c2

            lax.fori_loop(0, BLK, srow, 0)
            pltpu.sync_copy(rows, acc.at[dbuf], add=True)
            return carry

        lax.fori_loop(0, NBLK, blk, 0)
        plsc.subcore_barrier()

        @pl.when(sid < 15)
        def _():
            pltpu.sync_copy(acc.at[pl.ds(sid * R0, R0)],
                            out_hbm.at[cid, pl.ds(sid * R0, R0)])

        @pl.when(sid == 15)
        def _():
            pltpu.sync_copy(acc.at[pl.ds(15 * R0, R15)],
                            out_hbm.at[cid, pl.ds(15 * R0, R15)])

    return chunk


def _pack16(att):
    H, Cc = att.shape
    F = H * Cc
    m = jnp.zeros((H, Cc, 128), jnp.float32)
    for h in range(H):
        m = m.at[h, :, h].set(att[h])
    return m.reshape(F, 128)


def _rmat(H, F):
    Cc = F // H
    r = jnp.zeros((8, F), jnp.float32)
    for h in range(H):
        r = r.at[h, h * Cc:(h + 1) * Cc].set(1.0)
    return r


def _gat_layer(x, src_p, dst_p, zero_nf, W, att_s, att_d, bias, H,
               E_real, EPAD, g=None, be=None):
    N = x.shape[0]
    F = W.shape[1]
    C = F // BLK
    h, ps, pd = _pre(x, W, _pack16(att_s), _pack16(att_d))
    e, dpart = _make_epass(H, N, E_real, EPAD)(
        ps, pd, src_p, dst_p, zero_nf)
    parts = [_make_chunk(N, EPAD, c // 2)(
        h[:, c * BLK:(c + 1) * BLK], e, src_p, dst_p, zero_nf)
        for c in range(C)]
    np0 = jnp.concatenate([p[0] for p in parts], axis=1)
    np1 = jnp.concatenate([p[1] for p in parts], axis=1)
    dp0 = dpart[0][:, :8]
    dp1 = dpart[1][:, :8]
    return _post(np0, np1, dp0, dp1, _rmat(H, F), bias, g, be)


def kernel(x, edge_index, W1, a1s, a1d, b1, g1, be1, W2, a2s, a2d, b2,
           g2, be2, W3, a3s, a3d, b3):
    N = x.shape[0]
    E = edge_index.shape[1]
    E_real = E + N
    EPAD = -(-E_real // (NW * BLK)) * (NW * BLK)
    pad = EPAD - E_real
    loop = jnp.arange(N, dtype=edge_index.dtype)
    padv = jnp.arange(pad, dtype=edge_index.dtype)
    src_p = jnp.concatenate([edge_index[0], loop,
                             jnp.zeros((pad,), edge_index.dtype)])
    dst_p = jnp.concatenate([edge_index[1], loop, padv % N])
    zero_nf = jnp.zeros((N, BLK), jnp.float32)

    h = _gat_layer(x, src_p, dst_p, zero_nf, W1, a1s, a1d, b1, 4,
                   E_real, EPAD, g1, be1)
    h = _gat_layer(h, src_p, dst_p, zero_nf, W2, a2s, a2d, b2, 2,
                   E_real, EPAD, g2, be2)
    return _gat_layer(h, src_p, dst_p, zero_nf, W3, a3s, a3d, b3, 1,
                      E_real, EPAD)
